# baseline (speedup 1.0000x reference)
import functools

import jax
import jax.numpy as jnp
from jax import lax
from jax.experimental import pallas as pl
from jax.experimental.pallas import tpu as pltpu
from jax.experimental.pallas import tpu_sc as plsc

NC = 2
NS = 16
CW = 128
K0 = 48
K1 = 112


def _mesh():
    return plsc.VectorSubcoreMesh(core_axis_name="c", subcore_axis_name="s",
                                  num_cores=NC, num_subcores=NS)


def _zero_fill(ref, n_words):
    def body(k, _):
        ref[pl.ds(k * 16, 16)] = jnp.zeros((16,), jnp.float32)
        return 0
    lax.fori_loop(0, n_words // 16, body, 0, unroll=4)


def _make_deg_kernel(n_pad, acc_rows, deg_rows_per_tile):
    stripe = acc_rows // NS

    @functools.partial(
        pl.kernel,
        out_type=jax.ShapeDtypeStruct((NC, n_pad), jnp.float32),
        mesh=_mesh(),
        scratch_types=[
            pltpu.VMEM_SHARED((acc_rows,), jnp.float32),
            pltpu.VMEM((deg_rows_per_tile, CW), jnp.int32),
            pltpu.VMEM((CW,), jnp.float32),
            pltpu.VMEM((stripe,), jnp.float32),
        ],
    )
    def deg_kernel(dst_hbm, out_hbm, deg_sh, dstv, ones, zbuf):
        c = lax.axis_index("c")
        s = lax.axis_index("s")
        wid = c * NS + s
        _zero_fill(zbuf, stripe)
        for k in range(CW // 16):
            ones[pl.ds(k * 16, 16)] = jnp.ones((16,), jnp.float32)
        pltpu.sync_copy(zbuf, deg_sh.at[pl.ds(s * stripe, stripe)])
        plsc.subcore_barrier()
        pltpu.sync_copy(dst_hbm.at[pl.ds(wid * deg_rows_per_tile,
                                         deg_rows_per_tile)], dstv)
        def chunk(j, _):
            pltpu.sync_copy(ones, deg_sh.at[dstv.at[j]], add=True)
            return 0
        lax.fori_loop(0, deg_rows_per_tile, chunk, 0)
        plsc.subcore_barrier()
        @pl.when(s == 0)
        def _():
            pltpu.sync_copy(deg_sh.at[pl.ds(0, n_pad)], out_hbm.at[c])

    return deg_kernel


def _make_conv_kernel(n_pad, d, acc_rows):
    stripe = acc_rows // NS
    out_stripe = n_pad // NS
    kmax = max(K0, K1)

    @functools.partial(
        pl.kernel,
        out_type=jax.ShapeDtypeStruct((NC, n_pad, d), jnp.float32),
        mesh=_mesh(),
        scratch_types=[
            pltpu.VMEM_SHARED((acc_rows, d), jnp.float32),
            pltpu.VMEM((kmax, CW), jnp.int32),
            pltpu.VMEM((kmax, CW), jnp.int32),
            pltpu.VMEM((CW, d), jnp.float32),
            pltpu.SemaphoreType.DMA,
        ],
    )
    def conv_kernel(h_hbm, src_hbm, dst_hbm, out_hbm,
                    acc_sh, srcv, dstv, rows, sem):
        c = lax.axis_index("c")
        s = lax.axis_index("s")
        base = jnp.where(c == 0, s * K0, NS * K0 + s * K1)
        nrows = jnp.where(c == 0, K0, K1)
        def zrow(k, _):
            rows[k // (d // 16), pl.ds((k % (d // 16)) * 16, 16)] = (
                jnp.zeros((16,), jnp.float32))
            return 0
        lax.fori_loop(0, CW * (d // 16), zrow, 0, unroll=4)
        for r in range(stripe // CW):
            pltpu.sync_copy(rows, acc_sh.at[pl.ds(s * stripe + r * CW, CW)])
        pltpu.sync_copy(src_hbm.at[pl.ds(base, kmax)], srcv)
        pltpu.sync_copy(dst_hbm.at[pl.ds(base, kmax)], dstv)
        plsc.subcore_barrier()
        def chunk(j, _):
            pltpu.async_copy(h_hbm.at[srcv.at[j]], rows, sem).wait()
            pltpu.sync_copy(rows, acc_sh.at[dstv.at[j]], add=True)
            return 0
        lax.fori_loop(0, nrows, chunk, 0)
        plsc.subcore_barrier()
        pltpu.sync_copy(acc_sh.at[pl.ds(s * out_stripe, out_stripe)],
                        out_hbm.at[c, pl.ds(s * out_stripe, out_stripe)])

    return conv_kernel


def _tc1_body(n, n_pad, deg_ref, x_ref, w_ref, dinv_ref, hp_ref):
    deg = deg_ref[0] + deg_ref[1] + 1.0
    dinv = lax.rsqrt(deg)
    dinv_ref[...] = dinv
    h = jnp.dot(x_ref[...], w_ref[...], preferred_element_type=jnp.float32)
    hp_ref[0:n, :] = h * dinv[0:n]
    hp_ref[n:n_pad, :] = jnp.zeros((n_pad - n, h.shape[1]), jnp.float32)


def _tc2_body(s_ref, hp_ref, dinv_ref, b_ref, g_ref, be_ref, w_ref,
              x1_ref, h2p_ref):
    dinv = dinv_ref[...]
    z = dinv * (s_ref[0] + s_ref[1] + hp_ref[...]) + b_ref[...]
    mu = jnp.mean(z, axis=-1, keepdims=True)
    var = jnp.mean((z - mu) ** 2, axis=-1, keepdims=True)
    zn = (z - mu) * lax.rsqrt(var + 1e-5) * g_ref[...] + be_ref[...]
    x1 = jnp.maximum(zn, 0.0)
    x1_ref[...] = x1
    h2p_ref[...] = jnp.dot(x1, w_ref[...],
                           preferred_element_type=jnp.float32) * dinv


def _tc3_body(n, s_ref, hp_ref, dinv_ref, b_ref, g_ref, be_ref, x1_ref,
              wf_ref, bf_ref, out_ref):
    dinv = dinv_ref[...]
    z = dinv * (s_ref[0] + s_ref[1] + hp_ref[...]) + b_ref[...]
    mu = jnp.mean(z, axis=-1, keepdims=True)
    var = jnp.mean((z - mu) ** 2, axis=-1, keepdims=True)
    zn = (z - mu) * lax.rsqrt(var + 1e-5) * g_ref[...] + be_ref[...]
    x2 = jnp.maximum(zn, 0.0) + x1_ref[...]
    out = jnp.dot(x2, wf_ref[...], preferred_element_type=jnp.float32)
    out_ref[...] = out[0:n, :] + bf_ref[...]


def kernel(x, edge_index, W1, b1, g1, be1, W2, b2, g2, be2, Wf, bf):
    n, d = x.shape
    h = W1.shape[1]
    e = edge_index.shape[1]

    n_pad = ((n + 1 + 127) // 128) * 128
    acc_rows = ((n_pad + NS * CW - 1) // (NS * CW)) * NS * CW

    conv_rows = NS * (K0 + K1)
    assert conv_rows * CW >= e, "edge split must cover all edges"
    kmax = max(K0, K1)
    need = conv_rows + (kmax - min(K0, K1))
    granule = NC * NS * 8
    arr_rows = ((need + granule - 1) // granule) * granule
    deg_rows_per_tile = arr_rows // (NC * NS)
    e_pad = arr_rows * CW

    src = edge_index[0]
    dst = edge_index[1]
    pad_src = jnp.full((e_pad - e,), n, dtype=edge_index.dtype)
    pad_dst = (n + jnp.arange(e_pad - e, dtype=edge_index.dtype)
               % jnp.asarray(acc_rows - n, dtype=edge_index.dtype))
    srcr = jnp.concatenate([src, pad_src]).reshape(arr_rows, CW)
    dstr = jnp.concatenate([dst, pad_dst]).reshape(arr_rows, CW)

    deg_kernel = _make_deg_kernel(n_pad, acc_rows, deg_rows_per_tile)
    conv_kernel = _make_conv_kernel(n_pad, d, acc_rows)

    deg2 = deg_kernel(dstr)
    deg3 = deg2.reshape(NC, n_pad, 1)

    dinv, h1p = pl.pallas_call(
        functools.partial(_tc1_body, n, n_pad),
        out_shape=(jax.ShapeDtypeStruct((n_pad, 1), jnp.float32),
                   jax.ShapeDtypeStruct((n_pad, h), jnp.float32)),
    )(deg3, x, W1)

    s1 = conv_kernel(h1p, srcr, dstr)

    x1, h2p = pl.pallas_call(
        _tc2_body,
        out_shape=(jax.ShapeDtypeStruct((n_pad, h), jnp.float32),
                   jax.ShapeDtypeStruct((n_pad, h), jnp.float32)),
    )(s1, h1p, dinv, b1, g1, be1, W2)

    s2 = conv_kernel(h2p, srcr, dstr)

    out = pl.pallas_call(
        functools.partial(_tc3_body, n),
        out_shape=jax.ShapeDtypeStruct((n, Wf.shape[1]), jnp.float32),
    )(s2, h2p, dinv, b2, g2, be2, x1, Wf, bf)
    return out


# speedup vs baseline: 1.4149x; 1.4149x over previous
import functools

import jax
import jax.numpy as jnp
from jax import lax
from jax.experimental import pallas as pl
from jax.experimental.pallas import tpu as pltpu
from jax.experimental.pallas import tpu_sc as plsc

NC = 2
NS = 16
CW = 128
K0 = 80
K1 = 80


def _mesh():
    return plsc.VectorSubcoreMesh(core_axis_name="c", subcore_axis_name="s",
                                  num_cores=NC, num_subcores=NS)


def _zero_fill(ref, n_words):
    def body(k, _):
        ref[pl.ds(k * 16, 16)] = jnp.zeros((16,), jnp.float32)
        return 0
    lax.fori_loop(0, n_words // 16, body, 0, unroll=4)


def _make_deg_kernel(n_pad, acc_rows, deg_rows_per_tile):
    stripe = acc_rows // NS

    @functools.partial(
        pl.kernel,
        out_type=jax.ShapeDtypeStruct((NC, n_pad), jnp.float32),
        mesh=_mesh(),
        scratch_types=[
            pltpu.VMEM_SHARED((acc_rows,), jnp.float32),
            pltpu.VMEM((deg_rows_per_tile, CW), jnp.int32),
            pltpu.VMEM((CW,), jnp.float32),
            pltpu.VMEM((stripe,), jnp.float32),
        ],
    )
    def deg_kernel(dst_hbm, out_hbm, deg_sh, dstv, ones, zbuf):
        c = lax.axis_index("c")
        s = lax.axis_index("s")
        wid = c * NS + s
        _zero_fill(zbuf, stripe)
        for k in range(CW // 16):
            ones[pl.ds(k * 16, 16)] = jnp.ones((16,), jnp.float32)
        pltpu.sync_copy(zbuf, deg_sh.at[pl.ds(s * stripe, stripe)])
        plsc.subcore_barrier()
        pltpu.sync_copy(dst_hbm.at[pl.ds(wid * deg_rows_per_tile,
                                         deg_rows_per_tile)], dstv)
        def chunk(j, _):
            pltpu.sync_copy(ones, deg_sh.at[dstv.at[j]], add=True)
            return 0
        lax.fori_loop(0, deg_rows_per_tile, chunk, 0)
        plsc.subcore_barrier()
        @pl.when(s == 0)
        def _():
            pltpu.sync_copy(deg_sh.at[pl.ds(0, n_pad)], out_hbm.at[c])

    return deg_kernel


def _make_conv_kernel(n_pad, d, acc_rows):
    stripe = acc_rows // NS
    out_stripe = n_pad // NS
    kmax = max(K0, K1)

    @functools.partial(
        pl.kernel,
        out_type=jax.ShapeDtypeStruct((NC, n_pad, d), jnp.float32),
        mesh=_mesh(),
        scratch_types=[
            pltpu.VMEM_SHARED((acc_rows, d), jnp.float32),
            pltpu.VMEM((8, CW), jnp.int32),
            pltpu.VMEM((8, CW), jnp.int32),
            pltpu.VMEM((CW, d), jnp.float32),
            pltpu.VMEM((CW, d), jnp.float32),
            pltpu.SemaphoreType.DMA,
            pltpu.SemaphoreType.DMA,
        ],
    )
    def conv_kernel(h_hbm, src_hbm, dst_hbm, out_hbm,
                    acc_sh, srcv, dstv, rows, rows1, sem, sem1):
        c = lax.axis_index("c")
        s = lax.axis_index("s")
        base = jnp.where(c == 0, s * K0, NS * K0 + s * K1)
        nrows = jnp.where(c == 0, K0, K1)
        def zrow(k, _):
            rows[k // (d // 16), pl.ds((k % (d // 16)) * 16, 16)] = (
                jnp.zeros((16,), jnp.float32))
            return 0
        lax.fori_loop(0, CW * (d // 16), zrow, 0, unroll=4)
        for r in range(stripe // CW):
            pltpu.sync_copy(rows, acc_sh.at[pl.ds(s * stripe + r * CW, CW)])
        plsc.subcore_barrier()
        def block(b, _):
            pltpu.sync_copy(src_hbm.at[pl.ds(base + b * 8, 8)], srcv)
            pltpu.sync_copy(dst_hbm.at[pl.ds(base + b * 8, 8)], dstv)
            for p in range(4):
                pltpu.async_copy(h_hbm.at[srcv.at[2 * p]], rows, sem)
                pltpu.async_copy(h_hbm.at[srcv.at[2 * p + 1]], rows1, sem1)
                pltpu.make_async_copy(h_hbm.at[srcv.at[2 * p]], rows,
                                      sem).wait()
                pltpu.make_async_copy(h_hbm.at[srcv.at[2 * p + 1]], rows1,
                                      sem1).wait()
                pltpu.sync_copy(rows, acc_sh.at[dstv.at[2 * p]], add=True)
                pltpu.sync_copy(rows1, acc_sh.at[dstv.at[2 * p + 1]],
                                add=True)
            return 0
        lax.fori_loop(0, nrows // 8, block, 0)
        plsc.subcore_barrier()
        pltpu.sync_copy(acc_sh.at[pl.ds(s * out_stripe, out_stripe)],
                        out_hbm.at[c, pl.ds(s * out_stripe, out_stripe)])

    return conv_kernel


def _tc1_body(n, n_pad, deg_ref, x_ref, w_ref, dinv_ref, hp_ref):
    deg = deg_ref[0] + deg_ref[1] + 1.0
    dinv = lax.rsqrt(deg)
    dinv_ref[...] = dinv
    h = jnp.dot(x_ref[...], w_ref[...], preferred_element_type=jnp.float32)
    hp_ref[0:n, :] = h * dinv[0:n]
    hp_ref[n:n_pad, :] = jnp.zeros((n_pad - n, h.shape[1]), jnp.float32)


def _tc2_body(s_ref, hp_ref, dinv_ref, b_ref, g_ref, be_ref, w_ref,
              x1_ref, h2p_ref):
    dinv = dinv_ref[...]
    z = dinv * (s_ref[0] + s_ref[1] + hp_ref[...]) + b_ref[...]
    mu = jnp.mean(z, axis=-1, keepdims=True)
    var = jnp.mean((z - mu) ** 2, axis=-1, keepdims=True)
    zn = (z - mu) * lax.rsqrt(var + 1e-5) * g_ref[...] + be_ref[...]
    x1 = jnp.maximum(zn, 0.0)
    x1_ref[...] = x1
    h2p_ref[...] = jnp.dot(x1, w_ref[...],
                           preferred_element_type=jnp.float32) * dinv


def _tc3_body(n, s_ref, hp_ref, dinv_ref, b_ref, g_ref, be_ref, x1_ref,
              wf_ref, bf_ref, out_ref):
    dinv = dinv_ref[...]
    z = dinv * (s_ref[0] + s_ref[1] + hp_ref[...]) + b_ref[...]
    mu = jnp.mean(z, axis=-1, keepdims=True)
    var = jnp.mean((z - mu) ** 2, axis=-1, keepdims=True)
    zn = (z - mu) * lax.rsqrt(var + 1e-5) * g_ref[...] + be_ref[...]
    x2 = jnp.maximum(zn, 0.0) + x1_ref[...]
    out = jnp.dot(x2, wf_ref[...], preferred_element_type=jnp.float32)
    out_ref[...] = out[0:n, :] + bf_ref[...]


def kernel(x, edge_index, W1, b1, g1, be1, W2, b2, g2, be2, Wf, bf):
    n, d = x.shape
    h = W1.shape[1]
    e = edge_index.shape[1]

    n_pad = ((n + 1 + 127) // 128) * 128
    acc_rows = ((n_pad + NS * CW - 1) // (NS * CW)) * NS * CW

    conv_rows = NS * (K0 + K1)
    assert conv_rows * CW >= e, "edge split must cover all edges"
    kmax = max(K0, K1)
    need = conv_rows + (kmax - min(K0, K1))
    granule = NC * NS * 8
    arr_rows = ((need + granule - 1) // granule) * granule
    deg_rows_per_tile = arr_rows // (NC * NS)
    e_pad = arr_rows * CW

    src = edge_index[0]
    dst = edge_index[1]
    pad_src = jnp.full((e_pad - e,), n, dtype=edge_index.dtype)
    pad_dst = (n + jnp.arange(e_pad - e, dtype=edge_index.dtype)
               % jnp.asarray(acc_rows - n, dtype=edge_index.dtype))
    srcr = jnp.concatenate([src, pad_src]).reshape(arr_rows, CW)
    dstr = jnp.concatenate([dst, pad_dst]).reshape(arr_rows, CW)

    deg_kernel = _make_deg_kernel(n_pad, acc_rows, deg_rows_per_tile)
    conv_kernel = _make_conv_kernel(n_pad, d, acc_rows)

    deg2 = deg_kernel(dstr)
    deg3 = deg2.reshape(NC, n_pad, 1)

    dinv, h1p = pl.pallas_call(
        functools.partial(_tc1_body, n, n_pad),
        out_shape=(jax.ShapeDtypeStruct((n_pad, 1), jnp.float32),
                   jax.ShapeDtypeStruct((n_pad, h), jnp.float32)),
    )(deg3, x, W1)

    s1 = conv_kernel(h1p, srcr, dstr)

    x1, h2p = pl.pallas_call(
        _tc2_body,
        out_shape=(jax.ShapeDtypeStruct((n_pad, h), jnp.float32),
                   jax.ShapeDtypeStruct((n_pad, h), jnp.float32)),
    )(s1, h1p, dinv, b1, g1, be1, W2)

    s2 = conv_kernel(h2p, srcr, dstr)

    out = pl.pallas_call(
        functools.partial(_tc3_body, n),
        out_shape=jax.ShapeDtypeStruct((n, Wf.shape[1]), jnp.float32),
    )(s2, h2p, dinv, b2, g2, be2, x1, Wf, bf)
    return out
